# Initial kernel scaffold; baseline (speedup 1.0000x reference)
import jax
import jax.numpy as jnp
from jax.experimental import pallas as pl


def kernel(center_id, pos_context_id, neg_context_ids, W_in, W_out):
    raise NotImplementedError("write your pallas kernel here")


# speedup vs baseline: 4.1157x; 4.1157x over previous
import functools

import jax
import jax.numpy as jnp
from jax import lax
from jax.experimental import pallas as pl
from jax.experimental.pallas import tpu as pltpu
from jax.experimental.pallas import tpu_sc as plsc

B = 16384
D = 64
NNEG = 20
NC = 2
NS = 16
NW = NC * NS
RPW = B // NW
CH = 32
NCHUNK = RPW // CH
CROWS = CH * NNEG
IW = 128


def _sc_scores(cid2, pid2, nid2, w_in, w_out):
  mesh = plsc.VectorSubcoreMesh(core_axis_name="c", subcore_axis_name="s")

  @functools.partial(
      pl.kernel,
      mesh=mesh,
      compiler_params=pltpu.CompilerParams(
          needs_layout_passes=False, use_tc_tiling_on_sc=False),
      out_type=[
          jax.ShapeDtypeStruct((B,), jnp.float32),
          jax.ShapeDtypeStruct((B,), jnp.float32),
      ],
      scratch_types=[
          pltpu.VMEM((RPW // IW, IW), jnp.int32),
          pltpu.VMEM((RPW // IW, IW), jnp.int32),
          pltpu.VMEM((RPW * NNEG // IW, IW), jnp.int32),
          pltpu.VMEM((RPW, D), jnp.float32),
          pltpu.VMEM((2, CROWS, D), jnp.float32),
          pltpu.VMEM((RPW,), jnp.float32),
          pltpu.VMEM((RPW,), jnp.float32),
          pltpu.SemaphoreType.DMA,
          pltpu.SemaphoreType.DMA,
          pltpu.SemaphoreType.DMA,
      ],
  )
  def k(cid_hbm, pid_hbm, nid_hbm, win_hbm, wout_hbm, pos_out, neg_out,
        idx_c, idx_p, idx_n, cen, nbuf, pos_s, neg_s, sem_c, sem_a, sem_b):
    wid = lax.axis_index("s") * NC + lax.axis_index("c")
    crpw = RPW // IW
    nrpw = RPW * NNEG // IW
    nrpc = CROWS // IW

    pltpu.sync_copy(cid_hbm.at[pl.ds(wid * crpw, crpw)], idx_c)
    pltpu.sync_copy(pid_hbm.at[pl.ds(wid * crpw, crpw)], idx_p)
    pltpu.sync_copy(nid_hbm.at[pl.ds(wid * nrpw, nrpw)], idx_n)

    cen_cps = [
        pltpu.async_copy(win_hbm.at[idx_c.at[m]], cen.at[pl.ds(m * IW, IW)],
                         sem_c)
        for m in range(crpw)
    ]
    pos_cps = [
        pltpu.async_copy(wout_hbm.at[idx_p.at[m]],
                         nbuf.at[1, pl.ds(m * IW, IW)], sem_b)
        for m in range(crpw)
    ]

    def issue_chunk(c, sem):
      return [
          pltpu.async_copy(wout_hbm.at[idx_n.at[c * nrpc + m]],
                           nbuf.at[c % 2, pl.ds(m * IW, IW)], sem)
          for m in range(nrpc)
      ]

    cps0 = issue_chunk(0, sem_a)
    for cp in cen_cps:
      cp.wait()
    for cp in pos_cps:
      cp.wait()

    lanes = lax.iota(jnp.int32, 16)
    zero16 = jnp.zeros((16,), jnp.float32)

    def splat(x):
      return jnp.full((16,), x, jnp.int32)

    def pos_group(g, _):
      rows = g * 16 + lanes
      def dbody(d, acc):
        dcol = splat(d)
        cv = plsc.load_gather(cen, [rows, dcol])
        pv = plsc.load_gather(nbuf, [splat(1), rows, dcol])
        return acc + cv * pv
      pos_s[pl.ds(g * 16, 16)] = lax.fori_loop(0, D, dbody, zero16)
      return 0
    lax.fori_loop(0, RPW // 16, pos_group, 0)

    pending = cps0
    for c in range(NCHUNK):
      bi = c % 2
      nxt = []
      if c + 1 < NCHUNK:
        nxt = issue_chunk(c + 1, sem_b if bi == 0 else sem_a)
      for cp in pending:
        cp.wait()
      pending = nxt

      def grp(g, _, c=c, bi=bi):
        rows = c * CH + g * 16 + lanes
        lrow = (g * 16 + lanes) * NNEG
        def dbody(d, acc):
          dcol = splat(d)
          cv = plsc.load_gather(cen, [rows, dcol])
          for j in range(NNEG):
            nv = plsc.load_gather(nbuf, [splat(bi), lrow + j, dcol])
            acc = acc + cv * nv
          return acc
        neg_s[pl.ds(c * CH + g * 16, 16)] = lax.fori_loop(0, D, dbody, zero16)
        return 0
      lax.fori_loop(0, CH // 16, grp, 0)

    base = wid * RPW
    pltpu.sync_copy(pos_s, pos_out.at[pl.ds(base, RPW)])
    pltpu.sync_copy(neg_s, neg_out.at[pl.ds(base, RPW)])

  return k(cid2, pid2, nid2, w_in, w_out)


def _tc_loss(pos2d, neg2d):
  def body(p_ref, n_ref, o_ref):
    p = p_ref[...]
    m = -n_ref[...]
    lsp = jnp.minimum(p, 0.0) - jnp.log1p(jnp.exp(-jnp.abs(p)))
    lsn = jnp.minimum(m, 0.0) - jnp.log1p(jnp.exp(-jnp.abs(m)))
    o_ref[0, 0] = -(jnp.sum(lsp) + jnp.sum(lsn))

  out = pl.pallas_call(
      body,
      out_shape=jax.ShapeDtypeStruct((1, 1), jnp.float32),
      out_specs=pl.BlockSpec(memory_space=pltpu.SMEM),
  )(pos2d, neg2d)
  return out[0, 0]


def kernel(center_id, pos_context_id, neg_context_ids, W_in, W_out):
  cid2 = center_id.astype(jnp.int32).reshape(B // IW, IW)
  pid2 = pos_context_id.astype(jnp.int32).reshape(B // IW, IW)
  nid2 = neg_context_ids.astype(jnp.int32).reshape(B * NNEG // IW, IW)
  pos_score, neg_score = _sc_scores(cid2, pid2, nid2, W_in, W_out)
  return _tc_loss(pos_score.reshape(128, 128), neg_score.reshape(128, 128))
